# baseline (speedup 1.0000x reference)
import functools

import jax
import jax.numpy as jnp
from jax import lax
from jax.experimental import pallas as pl
from jax.experimental.pallas import tpu as pltpu
from jax.experimental.pallas import tpu_sc as plsc

B, C, N, K, OUT = 4, 128, 4096, 16, 128
RT = 512
R3 = 256
NEG_BIG = -3.0e38
CNT = B * N * K

_NW = 32
_CHUNK = 128


def _prep_kernel(x_ref, wyb_ref, wc_ref, b1_ref, y_ref, hc_ref, xx_ref):
    xb = x_ref[0]
    y = lax.dot_general(xb, wyb_ref[...], (((0,), (1,)), ((), ())),
                        preferred_element_type=jnp.float32)
    hc = lax.dot_general(xb, wc_ref[...], (((0,), (1,)), ((), ())),
                         preferred_element_type=jnp.float32)
    y_ref[0] = y
    hc_ref[0] = hc + b1_ref[...]
    xx_ref[0] = jnp.sum(xb * xb, axis=0, keepdims=True)


def _prep(x, w1b, wc, b1r):
    return pl.pallas_call(
        _prep_kernel,
        grid=(B,),
        in_specs=[
            pl.BlockSpec((1, C, N), lambda b: (b, 0, 0)),
            pl.BlockSpec((OUT, C), lambda b: (0, 0)),
            pl.BlockSpec((OUT, C), lambda b: (0, 0)),
            pl.BlockSpec((1, OUT), lambda b: (0, 0)),
        ],
        out_specs=[
            pl.BlockSpec((1, N, OUT), lambda b: (b, 0, 0)),
            pl.BlockSpec((1, N, OUT), lambda b: (b, 0, 0)),
            pl.BlockSpec((1, 1, N), lambda b: (b, 0, 0)),
        ],
        out_shape=[
            jax.ShapeDtypeStruct((B, N, OUT), jnp.float32),
            jax.ShapeDtypeStruct((B, N, OUT), jnp.float32),
            jax.ShapeDtypeStruct((B, 1, N), jnp.float32),
        ],
    )(x, w1b, wc, b1r)


def _knn_kernel(xr_ref, xall_ref, xxc_ref, idx_ref):
    xr = xr_ref[...]
    xa = xall_ref[...]
    xxc = xxc_ref[...]
    dot = lax.dot_general(xr, xa, (((0,), (0,)), ((), ())),
                          preferred_element_type=jnp.float32)
    ones = jnp.ones((C, 1), jnp.float32)
    xxr = lax.dot_general(xr * xr, ones, (((0,), (0,)), ((), ())),
                          preferred_element_type=jnp.float32)
    neg = 2.0 * dot - (xxr + xxc)
    iot = lax.broadcasted_iota(jnp.int32, (RT, N), 1)
    cols = []
    for _ in range(K):
        m = jnp.max(neg, axis=1, keepdims=True)
        eq = neg == m
        am = jnp.min(jnp.where(eq, iot, N), axis=1, keepdims=True)
        cols.append(am)
        neg = jnp.where(eq, NEG_BIG, neg)
    idx_ref[...] = jnp.concatenate(cols, axis=1)


def _knn_b(xb, xxb):
    return pl.pallas_call(
        _knn_kernel,
        grid=(N // RT,),
        in_specs=[
            pl.BlockSpec((C, RT), lambda j: (0, j)),
            pl.BlockSpec((C, N), lambda j: (0, 0)),
            pl.BlockSpec((1, N), lambda j: (0, 0)),
        ],
        out_specs=pl.BlockSpec((RT, K), lambda j: (j, 0)),
        out_shape=jax.ShapeDtypeStruct((N, K), jnp.int32),
    )(xb, xb, xxb)


def _sc_gather_body(y_hbm, idx_hbm, out_hbm, idx_v, rows_v, sem):
    wid = lax.axis_index("s") * 2 + lax.axis_index("c")
    per = (N * K) // _NW
    base0 = wid * per

    def body(i, carry):
        base = base0 + i * _CHUNK
        pltpu.sync_copy(idx_hbm.at[pl.ds(base, _CHUNK)], idx_v)
        pltpu.async_copy(y_hbm.at[idx_v], rows_v, sem).wait()
        pltpu.sync_copy(rows_v, out_hbm.at[pl.ds(base, _CHUNK)])
        return carry

    lax.fori_loop(0, per // _CHUNK, body, 0)


def _sc_gather_b(y2d, idxg):
    mesh = plsc.VectorSubcoreMesh(core_axis_name="c", subcore_axis_name="s")
    fn = functools.partial(
        pl.kernel,
        out_type=jax.ShapeDtypeStruct((N * K, OUT), jnp.float32),
        mesh=mesh,
        scratch_types=[
            pltpu.VMEM((_CHUNK,), jnp.int32),
            pltpu.VMEM((_CHUNK, OUT), jnp.float32),
            pltpu.SemaphoreType.DMA,
        ],
    )(_sc_gather_body)
    return fn(y2d, idxg)


def _stats1_kernel(g_ref, hc_ref, st_ref):
    step = pl.program_id(0)
    hc = hc_ref[...]
    acc = jnp.zeros((R3, OUT), jnp.float32)
    acc2 = jnp.zeros((R3, OUT), jnp.float32)
    for k in range(K):
        h = g_ref[k] + hc
        acc = acc + h
        acc2 = acc2 + h * h

    @pl.when(step == 0)
    def _():
        st_ref[...] = jnp.zeros_like(st_ref)

    st_ref[0:1, :] += jnp.sum(acc, axis=0, keepdims=True)
    st_ref[1:2, :] += jnp.sum(acc2, axis=0, keepdims=True)


def _stats1_b(g3, hc2):
    return pl.pallas_call(
        _stats1_kernel,
        grid=(N // R3,),
        in_specs=[
            pl.BlockSpec((K, R3, OUT), lambda i: (0, i, 0)),
            pl.BlockSpec((R3, OUT), lambda i: (i, 0)),
        ],
        out_specs=pl.BlockSpec((2, OUT), lambda i: (0, 0)),
        out_shape=jax.ShapeDtypeStruct((2, OUT), jnp.float32),
    )(g3, hc2)


def _main_kernel(g_ref, hc_ref, s1_ref, t1_ref, w2t_ref,
                 hmax_ref, hmin_ref, st_ref):
    step = pl.program_id(0)
    hcs = hc_ref[...] * s1_ref[...] + t1_ref[...]
    s1 = s1_ref[...]
    w2t = w2t_ref[...]
    acc = jnp.zeros((R3, OUT), jnp.float32)
    acc2 = jnp.zeros((R3, OUT), jnp.float32)
    hm = None
    hn = None
    for k in range(K):
        a = jnp.maximum(g_ref[k] * s1 + hcs, 0.0)
        h2 = lax.dot_general(a, w2t, (((1,), (0,)), ((), ())),
                             preferred_element_type=jnp.float32)
        acc = acc + h2
        acc2 = acc2 + h2 * h2
        hm = h2 if hm is None else jnp.maximum(hm, h2)
        hn = h2 if hn is None else jnp.minimum(hn, h2)
    hmax_ref[...] = hm
    hmin_ref[...] = hn

    @pl.when(step == 0)
    def _():
        st_ref[...] = jnp.zeros_like(st_ref)

    st_ref[0:1, :] += jnp.sum(acc, axis=0, keepdims=True)
    st_ref[1:2, :] += jnp.sum(acc2, axis=0, keepdims=True)


def _main_b(g3, hc2, s1r, t1r, w2t):
    return pl.pallas_call(
        _main_kernel,
        grid=(N // R3,),
        in_specs=[
            pl.BlockSpec((K, R3, OUT), lambda i: (0, i, 0)),
            pl.BlockSpec((R3, OUT), lambda i: (i, 0)),
            pl.BlockSpec((1, OUT), lambda i: (0, 0)),
            pl.BlockSpec((1, OUT), lambda i: (0, 0)),
            pl.BlockSpec((OUT, OUT), lambda i: (0, 0)),
        ],
        out_specs=[
            pl.BlockSpec((R3, OUT), lambda i: (i, 0)),
            pl.BlockSpec((R3, OUT), lambda i: (i, 0)),
            pl.BlockSpec((2, OUT), lambda i: (0, 0)),
        ],
        out_shape=[
            jax.ShapeDtypeStruct((N, OUT), jnp.float32),
            jax.ShapeDtypeStruct((N, OUT), jnp.float32),
            jax.ShapeDtypeStruct((2, OUT), jnp.float32),
        ],
    )(g3, hc2, s1r, t1r, w2t)


def _epi_kernel(hmax_ref, hmin_ref, s2_ref, t2_ref, out_ref):
    s2 = s2_ref[...]
    t2 = t2_ref[...]
    sel = jnp.where(s2 >= 0.0, hmax_ref[...], hmin_ref[...])
    out_ref[...] = jnp.maximum(sel * s2 + t2, 0.0)


def _epi_b(hmax, hmin, s2r, t2r):
    RE = 512
    return pl.pallas_call(
        _epi_kernel,
        grid=(N // RE,),
        in_specs=[
            pl.BlockSpec((RE, OUT), lambda i: (i, 0)),
            pl.BlockSpec((RE, OUT), lambda i: (i, 0)),
            pl.BlockSpec((1, OUT), lambda i: (0, 0)),
            pl.BlockSpec((1, OUT), lambda i: (0, 0)),
        ],
        out_specs=pl.BlockSpec((RE, OUT), lambda i: (i, 0)),
        out_shape=jax.ShapeDtypeStruct((N, OUT), jnp.float32),
    )(hmax, hmin, s2r, t2r)


def kernel(x, W1, b1, g1, be1, W2, b2, g2, be2):
    w1b = W1[:, C:]
    wc = W1[:, :C] - w1b
    y3, hc3, xx3 = _prep(x, w1b, wc, b1.reshape(1, OUT))
    y2d = y3.reshape(B * N, OUT)

    gs = []
    for b in range(B):
        idx_b = _knn_b(x[b], xx3[b])
        idxg_b = jnp.transpose(idx_b + b * N, (1, 0)).reshape(N * K)
        gs.append(_sc_gather_b(y2d, idxg_b).reshape(K, N, OUT))

    st1 = sum(_stats1_b(gs[b], hc3[b]) for b in range(B))
    mu1 = st1[0] / CNT
    var1 = st1[1] / CNT - mu1 * mu1
    s1 = g1 / jnp.sqrt(var1 + 1e-5)
    t1 = be1 - mu1 * s1

    s1r = s1.reshape(1, OUT)
    t1r = t1.reshape(1, OUT)
    w2t = W2.T
    res = [_main_b(gs[b], hc3[b], s1r, t1r, w2t) for b in range(B)]
    st2 = sum(r[2] for r in res)
    e2 = st2[0] / CNT
    var2 = st2[1] / CNT - e2 * e2
    s2 = g2 / jnp.sqrt(var2 + 1e-5)
    t2 = be2 - e2 * s2

    s2r = s2.reshape(1, OUT)
    t2r = t2.reshape(1, OUT)
    outs = [_epi_b(res[b][0], res[b][1], s2r, t2r) for b in range(B)]
    return jnp.stack(outs, axis=0).transpose(0, 2, 1)


# speedup vs baseline: 1.1192x; 1.1192x over previous
import functools

import jax
import jax.numpy as jnp
from jax import lax
from jax.experimental import pallas as pl
from jax.experimental.pallas import tpu as pltpu
from jax.experimental.pallas import tpu_sc as plsc

B, C, N, K, OUT = 4, 128, 4096, 16, 128
RT = 256
R3 = 256
NEG_BIG = -3.0e38
CNT = B * N * K

_NW = 32
_CHUNK = 128


def _prep_kernel(x_ref, wyb_ref, wc_ref, b1_ref, y_ref, hc_ref, xx_ref):
    xb = x_ref[0]
    y = lax.dot_general(xb, wyb_ref[...], (((0,), (1,)), ((), ())),
                        preferred_element_type=jnp.float32)
    hc = lax.dot_general(xb, wc_ref[...], (((0,), (1,)), ((), ())),
                         preferred_element_type=jnp.float32)
    y_ref[0] = y
    hc_ref[0] = hc + b1_ref[...]
    xx_ref[0] = jnp.sum(xb * xb, axis=0, keepdims=True)


def _prep(x, w1b, wc, b1r):
    return pl.pallas_call(
        _prep_kernel,
        grid=(B,),
        in_specs=[
            pl.BlockSpec((1, C, N), lambda b: (b, 0, 0)),
            pl.BlockSpec((OUT, C), lambda b: (0, 0)),
            pl.BlockSpec((OUT, C), lambda b: (0, 0)),
            pl.BlockSpec((1, OUT), lambda b: (0, 0)),
        ],
        out_specs=[
            pl.BlockSpec((1, N, OUT), lambda b: (b, 0, 0)),
            pl.BlockSpec((1, N, OUT), lambda b: (b, 0, 0)),
            pl.BlockSpec((1, 1, N), lambda b: (b, 0, 0)),
        ],
        out_shape=[
            jax.ShapeDtypeStruct((B, N, OUT), jnp.float32),
            jax.ShapeDtypeStruct((B, N, OUT), jnp.float32),
            jax.ShapeDtypeStruct((B, 1, N), jnp.float32),
        ],
    )(x, w1b, wc, b1r)


def _knn_kernel(xr_ref, xall_ref, xxc_ref, idx_ref):
    xr = xr_ref[...]
    xa = xall_ref[...]
    xxc = xxc_ref[...]
    dot = lax.dot_general(xr, xa, (((0,), (0,)), ((), ())),
                          preferred_element_type=jnp.float32)
    ones = jnp.ones((C, 1), jnp.float32)
    xxr = lax.dot_general(xr * xr, ones, (((0,), (0,)), ((), ())),
                          preferred_element_type=jnp.float32)
    neg = 2.0 * dot - (xxr + xxc)
    bits = lax.bitcast_convert_type(neg, jnp.int32)
    si = jnp.where(bits < 0, bits ^ 0x7FFFFFFF, bits)
    iot = lax.broadcasted_iota(jnp.int32, (RT, N), 1)
    nch = N // 128
    key = (si & (-32)) | ((nch - 1) - (iot >> 7))
    iot128 = lax.broadcasted_iota(jnp.int32, (RT, 128), 1)
    cols = []
    for _ in range(K):
        lmk = key[:, 0:128]
        for c in range(1, nch):
            lmk = jnp.maximum(lmk, key[:, c * 128:(c + 1) * 128])
        mk = jnp.max(lmk, axis=1, keepdims=True)
        cj = (nch - 1) - (mk & (nch - 1))
        le = jnp.min(jnp.where(lmk == mk, iot128, 128), axis=1,
                     keepdims=True)
        am = cj * 128 + le
        cols.append(am)
        key = jnp.where(iot == am, -2147483648, key)
    idx_ref[...] = jnp.concatenate(cols, axis=1)


def _knn_b(xb, xxb):
    return pl.pallas_call(
        _knn_kernel,
        grid=(N // RT,),
        in_specs=[
            pl.BlockSpec((C, RT), lambda j: (0, j)),
            pl.BlockSpec((C, N), lambda j: (0, 0)),
            pl.BlockSpec((1, N), lambda j: (0, 0)),
        ],
        out_specs=pl.BlockSpec((RT, K), lambda j: (j, 0)),
        out_shape=jax.ShapeDtypeStruct((N, K), jnp.int32),
    )(xb, xb, xxb)


def _sc_gather_body(y_hbm, idx_hbm, out_hbm, idx_v, rows_v, sem):
    wid = lax.axis_index("s") * 2 + lax.axis_index("c")
    per = (N * K) // _NW
    base0 = wid * per

    def body(i, carry):
        base = base0 + i * _CHUNK
        pltpu.sync_copy(idx_hbm.at[pl.ds(base, _CHUNK)], idx_v)
        pltpu.async_copy(y_hbm.at[idx_v], rows_v, sem).wait()
        pltpu.sync_copy(rows_v, out_hbm.at[pl.ds(base, _CHUNK)])
        return carry

    lax.fori_loop(0, per // _CHUNK, body, 0)


def _sc_gather_b(y2d, idxg):
    mesh = plsc.VectorSubcoreMesh(core_axis_name="c", subcore_axis_name="s")
    fn = functools.partial(
        pl.kernel,
        out_type=jax.ShapeDtypeStruct((N * K, OUT), jnp.float32),
        mesh=mesh,
        scratch_types=[
            pltpu.VMEM((_CHUNK,), jnp.int32),
            pltpu.VMEM((_CHUNK, OUT), jnp.float32),
            pltpu.SemaphoreType.DMA,
        ],
    )(_sc_gather_body)
    return fn(y2d, idxg)


def _stats1_kernel(g_ref, hc_ref, st_ref):
    step = pl.program_id(0)
    hc = hc_ref[...]
    acc = jnp.zeros((R3, OUT), jnp.float32)
    acc2 = jnp.zeros((R3, OUT), jnp.float32)
    for k in range(K):
        h = g_ref[k] + hc
        acc = acc + h
        acc2 = acc2 + h * h

    @pl.when(step == 0)
    def _():
        st_ref[...] = jnp.zeros_like(st_ref)

    st_ref[0:1, :] += jnp.sum(acc, axis=0, keepdims=True)
    st_ref[1:2, :] += jnp.sum(acc2, axis=0, keepdims=True)


def _stats1_b(g3, hc2):
    return pl.pallas_call(
        _stats1_kernel,
        grid=(N // R3,),
        in_specs=[
            pl.BlockSpec((K, R3, OUT), lambda i: (0, i, 0)),
            pl.BlockSpec((R3, OUT), lambda i: (i, 0)),
        ],
        out_specs=pl.BlockSpec((2, OUT), lambda i: (0, 0)),
        out_shape=jax.ShapeDtypeStruct((2, OUT), jnp.float32),
    )(g3, hc2)


def _main_kernel(g_ref, hc_ref, s1_ref, t1_ref, w2t_ref,
                 hmax_ref, hmin_ref, st_ref):
    step = pl.program_id(0)
    hcs = hc_ref[...] * s1_ref[...] + t1_ref[...]
    s1 = s1_ref[...]
    w2t = w2t_ref[...]
    acc = jnp.zeros((R3, OUT), jnp.float32)
    acc2 = jnp.zeros((R3, OUT), jnp.float32)
    hm = None
    hn = None
    for k in range(K):
        a = jnp.maximum(g_ref[k] * s1 + hcs, 0.0)
        h2 = lax.dot_general(a, w2t, (((1,), (0,)), ((), ())),
                             preferred_element_type=jnp.float32)
        acc = acc + h2
        acc2 = acc2 + h2 * h2
        hm = h2 if hm is None else jnp.maximum(hm, h2)
        hn = h2 if hn is None else jnp.minimum(hn, h2)
    hmax_ref[...] = hm
    hmin_ref[...] = hn

    @pl.when(step == 0)
    def _():
        st_ref[...] = jnp.zeros_like(st_ref)

    st_ref[0:1, :] += jnp.sum(acc, axis=0, keepdims=True)
    st_ref[1:2, :] += jnp.sum(acc2, axis=0, keepdims=True)


def _main_b(g3, hc2, s1r, t1r, w2t):
    return pl.pallas_call(
        _main_kernel,
        grid=(N // R3,),
        in_specs=[
            pl.BlockSpec((K, R3, OUT), lambda i: (0, i, 0)),
            pl.BlockSpec((R3, OUT), lambda i: (i, 0)),
            pl.BlockSpec((1, OUT), lambda i: (0, 0)),
            pl.BlockSpec((1, OUT), lambda i: (0, 0)),
            pl.BlockSpec((OUT, OUT), lambda i: (0, 0)),
        ],
        out_specs=[
            pl.BlockSpec((R3, OUT), lambda i: (i, 0)),
            pl.BlockSpec((R3, OUT), lambda i: (i, 0)),
            pl.BlockSpec((2, OUT), lambda i: (0, 0)),
        ],
        out_shape=[
            jax.ShapeDtypeStruct((N, OUT), jnp.float32),
            jax.ShapeDtypeStruct((N, OUT), jnp.float32),
            jax.ShapeDtypeStruct((2, OUT), jnp.float32),
        ],
    )(g3, hc2, s1r, t1r, w2t)


def _epi_kernel(hmax_ref, hmin_ref, s2_ref, t2_ref, out_ref):
    s2 = s2_ref[...]
    t2 = t2_ref[...]
    sel = jnp.where(s2 >= 0.0, hmax_ref[...], hmin_ref[...])
    out_ref[...] = jnp.maximum(sel * s2 + t2, 0.0)


def _epi_b(hmax, hmin, s2r, t2r):
    RE = 512
    return pl.pallas_call(
        _epi_kernel,
        grid=(N // RE,),
        in_specs=[
            pl.BlockSpec((RE, OUT), lambda i: (i, 0)),
            pl.BlockSpec((RE, OUT), lambda i: (i, 0)),
            pl.BlockSpec((1, OUT), lambda i: (0, 0)),
            pl.BlockSpec((1, OUT), lambda i: (0, 0)),
        ],
        out_specs=pl.BlockSpec((RE, OUT), lambda i: (i, 0)),
        out_shape=jax.ShapeDtypeStruct((N, OUT), jnp.float32),
    )(hmax, hmin, s2r, t2r)


def kernel(x, W1, b1, g1, be1, W2, b2, g2, be2):
    w1b = W1[:, C:]
    wc = W1[:, :C] - w1b
    y3, hc3, xx3 = _prep(x, w1b, wc, b1.reshape(1, OUT))
    y2d = y3.reshape(B * N, OUT)

    gs = []
    for b in range(B):
        idx_b = _knn_b(x[b], xx3[b])
        idxg_b = jnp.transpose(idx_b + b * N, (1, 0)).reshape(N * K)
        gs.append(_sc_gather_b(y2d, idxg_b).reshape(K, N, OUT))

    st1 = sum(_stats1_b(gs[b], hc3[b]) for b in range(B))
    mu1 = st1[0] / CNT
    var1 = st1[1] / CNT - mu1 * mu1
    s1 = g1 / jnp.sqrt(var1 + 1e-5)
    t1 = be1 - mu1 * s1

    s1r = s1.reshape(1, OUT)
    t1r = t1.reshape(1, OUT)
    w2t = W2.T
    res = [_main_b(gs[b], hc3[b], s1r, t1r, w2t) for b in range(B)]
    st2 = sum(r[2] for r in res)
    e2 = st2[0] / CNT
    var2 = st2[1] / CNT - e2 * e2
    s2 = g2 / jnp.sqrt(var2 + 1e-5)
    t2 = be2 - e2 * s2

    s2r = s2.reshape(1, OUT)
    t2r = t2.reshape(1, OUT)
    outs = [_epi_b(res[b][0], res[b][1], s2r, t2r) for b in range(B)]
    return jnp.stack(outs, axis=0).transpose(0, 2, 1)
